# baseline (speedup 1.0000x reference)
import jax
import jax.numpy as jnp
from jax.experimental import pallas as pl


def _chamfer_body(ti, x1_ref, x2t_ref, idx1_ref, idx2_ref):
    nb = x1_ref.shape[0]
    n = x1_ref.shape[1]
    m = x2t_ref.shape[2]
    jidx = jax.lax.broadcasted_iota(
        jnp.int32, (1, m), 1).astype(jnp.float32)
    iidx = jax.lax.broadcasted_iota(
        jnp.int32, (ti, 1), 0).astype(jnp.float32)

    for bi in range(nb):
        x2t = x2t_ref[bi]
        x2t2 = x2t + x2t
        b2 = jnp.sum(x2t * x2t, axis=0, keepdims=True)
        cmin_run = jnp.full((1, m), jnp.inf, jnp.float32)
        carg_run = jnp.zeros((1, m), jnp.float32)
        for t in range(n // ti):
            i0 = t * ti
            x1t = x1_ref[bi, pl.ds(i0, ti), :]
            a2 = jnp.sum(x1t * x1t, axis=1, keepdims=True)
            ab2 = jax.lax.dot_general(
                x1t, x2t2, (((1,), (0,)), ((), ())),
                preferred_element_type=jnp.float32)
            d = (a2 + b2) - ab2
            rmin = jnp.min(d, axis=1, keepdims=True)
            rarg = jnp.min(jnp.where(d == rmin, jidx, jnp.inf),
                           axis=1, keepdims=True)
            idx1_ref[bi, :, i0:i0 + ti] = rarg.astype(jnp.int32).reshape(1, ti)
            cmin = jnp.min(d, axis=0, keepdims=True)
            carg = jnp.min(jnp.where(d == cmin, iidx, jnp.inf),
                           axis=0, keepdims=True) + float(i0)
            take = cmin < cmin_run
            carg_run = jnp.where(take, carg, carg_run)
            cmin_run = jnp.where(take, cmin, cmin_run)

        idx2_ref[bi, :, :] = carg_run.astype(jnp.int32)


def _chamfer_batches(xyz1, xyz2):
    b, n, d = xyz1.shape
    m = xyz2.shape[1]
    ti = 512
    x2t = jnp.transpose(xyz2, (0, 2, 1))
    idx1, idx2 = pl.pallas_call(
        lambda *refs: _chamfer_body(ti, *refs),
        out_shape=[jax.ShapeDtypeStruct((b, 1, n), jnp.int32),
                   jax.ShapeDtypeStruct((b, 1, m), jnp.int32)],
    )(xyz1, x2t)
    return idx1.reshape(b, n), idx2.reshape(b, m)


def kernel(xyz1, xyz2):
    return _chamfer_batches(xyz1, xyz2)


# speedup vs baseline: 1.0867x; 1.0867x over previous
import jax
import jax.numpy as jnp
from jax.experimental import pallas as pl


def _chamfer_body(ti, x1_ref, x2t_ref, idx1_ref, idx2_ref):
    n = x1_ref.shape[1]
    m = x2t_ref.shape[2]
    x2t = x2t_ref[0]
    x2t2 = x2t + x2t
    b2 = jnp.sum(x2t * x2t, axis=0, keepdims=True)
    jidx = jax.lax.broadcasted_iota(
        jnp.int32, (1, m), 1).astype(jnp.float32)
    iidx = jax.lax.broadcasted_iota(
        jnp.int32, (ti, 1), 0).astype(jnp.float32)
    eye = jnp.equal(
        jax.lax.broadcasted_iota(jnp.int32, (ti, ti), 0),
        jax.lax.broadcasted_iota(jnp.int32, (ti, ti), 1)).astype(jnp.float32)

    cmin_run = jnp.full((1, m), jnp.inf, jnp.float32)
    carg_run = jnp.zeros((1, m), jnp.float32)
    for t in range(n // ti):
        i0 = t * ti
        x1t = x1_ref[0, pl.ds(i0, ti), :]
        a2 = jnp.sum(x1t * x1t, axis=1, keepdims=True)
        ab2 = jax.lax.dot_general(
            x1t, x2t2, (((1,), (0,)), ((), ())),
            preferred_element_type=jnp.float32)
        d = (a2 + b2) - ab2
        rmin = jnp.min(d, axis=1, keepdims=True)
        rarg = jnp.min(jnp.where(d == rmin, jidx, jnp.inf),
                       axis=1, keepdims=True)
        rarg_t = jax.lax.dot_general(
            rarg, eye, (((0,), (0,)), ((), ())),
            preferred_element_type=jnp.float32)
        idx1_ref[0, :, i0:i0 + ti] = rarg_t.astype(jnp.int32)
        cmin = jnp.min(d, axis=0, keepdims=True)
        carg = jnp.min(jnp.where(d == cmin, iidx, jnp.inf),
                       axis=0, keepdims=True) + float(i0)
        take = cmin < cmin_run
        carg_run = jnp.where(take, carg, carg_run)
        cmin_run = jnp.where(take, cmin, cmin_run)

    idx2_ref[0, :, :] = carg_run.astype(jnp.int32)


def _chamfer_batches(xyz1, xyz2):
    b, n, d = xyz1.shape
    m = xyz2.shape[1]
    ti = 512
    x2t = jnp.transpose(xyz2, (0, 2, 1))
    idx1, idx2 = pl.pallas_call(
        lambda *refs: _chamfer_body(ti, *refs),
        grid=(b,),
        in_specs=[pl.BlockSpec((1, n, d), lambda i: (i, 0, 0)),
                  pl.BlockSpec((1, d, m), lambda i: (i, 0, 0))],
        out_specs=[pl.BlockSpec((1, 1, n), lambda i: (i, 0, 0)),
                   pl.BlockSpec((1, 1, m), lambda i: (i, 0, 0))],
        out_shape=[jax.ShapeDtypeStruct((b, 1, n), jnp.int32),
                   jax.ShapeDtypeStruct((b, 1, m), jnp.int32)],
    )(xyz1, x2t)
    return idx1.reshape(b, n), idx2.reshape(b, m)


def kernel(xyz1, xyz2):
    return _chamfer_batches(xyz1, xyz2)
